# baseline (speedup 1.0000x reference)
import functools
import math

import jax
import jax.numpy as jnp
from jax import lax
from jax.experimental import pallas as pl
from jax.experimental.pallas import tpu as pltpu
from jax.experimental.pallas import tpu_sc as plsc

D = 64
SCALE = math.sqrt(D)
NC, NS, L = 2, 16, 16
NW = NC * NS
V = 1000000
B_ROWS = 4096
SEQ = 200
BPW = B_ROWS // NW

_SC_PARAMS = pltpu.CompilerParams(needs_layout_passes=False)


@functools.partial(
    pl.kernel,
    mesh=plsc.VectorSubcoreMesh(core_axis_name="c", subcore_axis_name="s"),
    compiler_params=_SC_PARAMS,
    out_type=jax.ShapeDtypeStruct((SEQ, D, B_ROWS), jnp.float32),
    scratch_types=[
        pltpu.VMEM((SEQ, BPW), jnp.int32),
        pltpu.VMEM((BPW,), jnp.int32),
        pltpu.VMEM((BPW,), jnp.int32),
        pltpu.VMEM((BPW, 2 * D), jnp.float32),
        pltpu.VMEM((BPW, 2 * D), jnp.float32),
        pltpu.VMEM((D, BPW), jnp.float32),
        pltpu.VMEM((D, BPW), jnp.float32),
        pltpu.SemaphoreType.DMA,
        pltpu.SemaphoreType.DMA,
        pltpu.SemaphoreType.DMA,
        pltpu.SemaphoreType.DMA,
    ],
)
def _gather_scaled(xt_hbm, table_hbm, out_hbm, idx_v, ix0, ix1,
                   buf0, buf1, sl0, sl1, g0, g1, s0, s1):
    c = lax.axis_index("c")
    s = lax.axis_index("s")
    wid = s * NC + c
    b0 = pl.multiple_of(wid * BPW, BPW)
    pltpu.sync_copy(xt_hbm.at[:, pl.ds(b0, BPW)], idx_v)
    lanes = lax.iota(jnp.int32, L)
    brows = [lanes + bb * L for bb in range(BPW // L)]
    ixs, bufs, slabs, gs, ss = (ix0, ix1), (buf0, buf1), (sl0, sl1), \
        (g0, g1), (s0, s1)

    def start_gather(t, b):
        def halve(g, carry):
            iv = idx_v[t, pl.ds(g * L, L)]
            ixs[b][pl.ds(g * L, L)] = lax.shift_right_logical(iv, 1)
            return carry

        lax.fori_loop(0, BPW // L, halve, 0)
        pltpu.async_copy(table_hbm.at[ixs[b]], bufs[b], gs[b])

    def wait_gather(b):
        pltpu.make_async_copy(table_hbm.at[pl.ds(0, BPW)], bufs[b],
                              gs[b]).wait()

    def start_out(t, b):
        pltpu.async_copy(slabs[b], out_hbm.at[t, :, pl.ds(b0, BPW)], ss[b])

    def wait_out(b):
        pltpu.make_async_copy(slabs[b], out_hbm.at[0, :, pl.ds(b0, BPW)],
                              ss[b]).wait()

    start_gather(0, 0)
    start_gather(1, 1)

    def pair(p, carry):
        for b in range(2):
            t = 2 * p + b
            wait_gather(b)

            @pl.when(t >= 2)
            def _():
                wait_out(b)

            for bb in range(BPW // L):
                iv = idx_v[t, pl.ds(bb * L, L)]
                pbase = (iv & 1) << 6
                for d0 in range(0, D, L):
                    vs = [plsc.load_gather(bufs[b],
                                           [brows[bb], pbase + (d0 + i)])
                          for i in range(L)]
                    for i in range(L):
                        slabs[b][d0 + i, pl.ds(bb * L, L)] = vs[i] * SCALE

            start_out(t, b)

            @pl.when(t + 2 < SEQ)
            def _():
                start_gather(t + 2, b)

        return carry

    lax.fori_loop(0, SEQ // 2, pair, 0)
    wait_out(0)
    wait_out(1)


def kernel(x, table):
    xt = x.T.astype(jnp.int32)
    t64 = table.reshape(V // 2, 2 * D)
    ok = _gather_scaled(xt, t64)
    return ok.transpose(2, 0, 1)
